# baseline (speedup 1.0000x reference)
import functools

import jax
import jax.numpy as jnp
from jax import lax
from jax.experimental import pallas as pl
from jax.experimental.pallas import tpu as pltpu
from jax.experimental.pallas import tpu_sc as plsc

N_USERS = 50000
N_ITEMS = 50000
N_TOTAL = N_USERS + N_ITEMS
EMB_DIM = 32
HALF = 16
N_EDGES = 1600000
N_HOPS = 3

NS = 16
K = 8
CHUNK = K * 128
STAGES = 98
EDGES_PER_TILE = STAGES * CHUNK
E_PAD = NS * EDGES_PER_TILE
N_PAD = 100096
ROWS_PER_TILE = N_PAD // NS

_mesh = plsc.VectorSubcoreMesh(core_axis_name="c", subcore_axis_name="s")


@functools.partial(
    pl.kernel,
    mesh=_mesh,
    out_type=[jax.ShapeDtypeStruct((2, N_PAD, HALF), jnp.float32)] * N_HOPS,
    compiler_params=pltpu.CompilerParams(use_tc_tiling_on_sc=False),
    scratch_types=[
        pltpu.VMEM((K, 128), jnp.int32),
        pltpu.VMEM((K, 128), jnp.int32),
        pltpu.VMEM((K, 128), jnp.float32),
        pltpu.VMEM((K, 128), jnp.int32),
        pltpu.VMEM((K, 128), jnp.int32),
        pltpu.VMEM((K, 128), jnp.float32),
        pltpu.VMEM((6, 128, HALF), jnp.float32),
        pltpu.SemaphoreType.DMA,
        pltpu.SemaphoreType.DMA,
        pltpu.SemaphoreType.DMA,
        pltpu.SemaphoreType.DMA,
        pltpu.SemaphoreType.DMA,
        pltpu.VMEM_SHARED((N_PAD, HALF), jnp.float32),
    ],
)
def _gcn(tab_hbm, row_hbm, col_hbm, val_hbm, zeros_hbm,
         out1, out2, out3,
         row_a, col_a, val_a, row_b, col_b, val_b, msg,
         esem, gsem0, gsem1, ssem0, ssem1, acc_sh):
    c = lax.axis_index("c")
    s = lax.axis_index("s")

    base128 = s * (STAGES * K)
    my_rows = pl.ds(s * ROWS_PER_TILE, ROWS_PER_TILE)

    def issue_edges(st, bufs):
        row_r, col_r, val_r = bufs
        pltpu.async_copy(row_hbm.at[pl.ds(st, K)], row_r, esem)
        pltpu.async_copy(col_hbm.at[pl.ds(st, K)], col_r, esem)
        pltpu.async_copy(val_hbm.at[pl.ds(st, K)], val_r, esem)

    def drain_edges(st, bufs):
        row_r, col_r, val_r = bufs
        pltpu.make_async_copy(row_hbm.at[pl.ds(st, K)], row_r, esem).wait()
        pltpu.make_async_copy(col_hbm.at[pl.ds(st, K)], col_r, esem).wait()
        pltpu.make_async_copy(val_hbm.at[pl.ds(st, K)], val_r, esem).wait()

    def scale(j, val_r):
        mb = msg.at[j % 6]

        def scale_group(g, _):
            vv = val_r[j, pl.ds(g * 16, 16)]
            base = g * 16
            for e in range(16):
                mb[base + e, :] = mb[base + e, :] * vv[e]
            return 0

        lax.fori_loop(0, 8, scale_group, 0)

    def stage_block(src, bufs):
        row_r, col_r, val_r = bufs
        gsems = (gsem0, gsem1)
        ssems = (ssem0, ssem1)
        g = [None] * K
        sc = [None] * K
        g[0] = pltpu.async_copy(src.at[col_r.at[0]], msg.at[0], gsem0)
        g[1] = pltpu.async_copy(src.at[col_r.at[1]], msg.at[1], gsem1)
        for j in range(K):
            g[j].wait()
            if j + 2 < K:
                g[j + 2] = pltpu.async_copy(
                    src.at[col_r.at[j + 2]], msg.at[(j + 2) % 6], gsems[j % 2])
            scale(j, val_r)
            if j >= 2:
                sc[j - 2].wait()
            sc[j] = pltpu.async_copy(
                msg.at[j % 6], acc_sh.at[row_r.at[j]], ssems[j % 2], add=True)
        sc[K - 2].wait()
        sc[K - 1].wait()

    bufs_a = (row_a, col_a, val_a)
    bufs_b = (row_b, col_b, val_b)

    pltpu.sync_copy(zeros_hbm, acc_sh.at[my_rows])
    issue_edges(base128, bufs_a)
    plsc.subcore_barrier()

    srcs = (tab_hbm, out1, out2)
    dsts = (out1, out2, out3)

    for h in range(N_HOPS):
        src = srcs[h].at[c]

        def pair_body(t, _, src=src):
            st0 = base128 + (2 * t) * K
            st1 = st0 + K
            st2 = st1 + K
            drain_edges(st0, bufs_a)
            issue_edges(st1, bufs_b)
            stage_block(src, bufs_a)
            drain_edges(st1, bufs_b)

            @pl.when(t + 1 < STAGES // 2)
            def _():
                issue_edges(st2, bufs_a)

            stage_block(src, bufs_b)
            return 0

        lax.fori_loop(0, STAGES // 2, pair_body, 0)

        if h + 1 < N_HOPS:
            issue_edges(base128, bufs_a)
        plsc.subcore_barrier()
        pltpu.sync_copy(acc_sh.at[my_rows], dsts[h].at[c, my_rows])
        if h + 1 < N_HOPS:
            pltpu.sync_copy(zeros_hbm, acc_sh.at[my_rows])
        plsc.subcore_barrier()


def kernel(user_embed, item_embed, edge_index, edge_vals):
    all_embed = jnp.concatenate([user_embed, item_embed], axis=0)
    all_embed = jnp.pad(all_embed, ((0, N_PAD - N_TOTAL), (0, 0)))
    tab = jnp.stack([all_embed[:, :HALF], all_embed[:, HALF:]])

    pad = E_PAD - N_EDGES
    row = jnp.concatenate([edge_index[0], jnp.zeros((pad,), edge_index.dtype)])
    col = jnp.concatenate([edge_index[1], jnp.zeros((pad,), edge_index.dtype)])
    val = jnp.concatenate([edge_vals, jnp.zeros((pad,), edge_vals.dtype)])
    row = row.reshape(-1, 128)
    col = col.reshape(-1, 128)
    val = val.reshape(-1, 128)
    zeros = jnp.zeros((ROWS_PER_TILE, HALF), jnp.float32)

    outs = _gcn(tab, row, col, val, zeros)
    tabs = [tab] + list(outs)

    embs = jnp.stack(
        [jnp.concatenate([t[0, :N_TOTAL], t[1, :N_TOTAL]], axis=-1) for t in tabs],
        axis=1,
    )
    return embs[:N_USERS], embs[N_USERS:]


# speedup vs baseline: 1.2352x; 1.2352x over previous
import functools

import jax
import jax.numpy as jnp
from jax import lax
from jax.experimental import pallas as pl
from jax.experimental.pallas import tpu as pltpu
from jax.experimental.pallas import tpu_sc as plsc

N_USERS = 50000
N_ITEMS = 50000
N_TOTAL = N_USERS + N_ITEMS
EMB_DIM = 32
HALF = 16
N_EDGES = 1600000
N_HOPS = 3

NS = 16
K = 8
CHUNK = K * 128
STAGES = 98
EDGES_PER_TILE = STAGES * CHUNK
E_PAD = NS * EDGES_PER_TILE
N_PAD = 100096
ROWS_PER_TILE = N_PAD // NS

_mesh = plsc.VectorSubcoreMesh(core_axis_name="c", subcore_axis_name="s")


@functools.partial(
    pl.kernel,
    mesh=_mesh,
    out_type=jax.ShapeDtypeStruct((2, N_PAD, HALF), jnp.float32),
    compiler_params=pltpu.CompilerParams(use_tc_tiling_on_sc=False),
    scratch_types=[
        pltpu.VMEM((K, 128), jnp.int32),
        pltpu.VMEM((K, 128), jnp.int32),
        pltpu.VMEM((K, 128), jnp.float32),
        pltpu.VMEM((K, 128), jnp.int32),
        pltpu.VMEM((K, 128), jnp.int32),
        pltpu.VMEM((K, 128), jnp.float32),
        pltpu.VMEM((6, 128, HALF), jnp.float32),
        pltpu.SemaphoreType.DMA,
        pltpu.SemaphoreType.DMA,
        pltpu.SemaphoreType.DMA,
        pltpu.SemaphoreType.DMA,
        pltpu.SemaphoreType.DMA,
        pltpu.SemaphoreType.DMA,
        pltpu.VMEM_SHARED((N_PAD, HALF), jnp.float32),
    ],
)
def _hop(tab_hbm, row_hbm, col_hbm, val_hbm, zeros_hbm, out_hbm,
         row_a, col_a, val_a, row_b, col_b, val_b, msg,
         esem, gsem0, gsem1, gsem2, ssem0, ssem1, acc_sh):
    c = lax.axis_index("c")
    s = lax.axis_index("s")

    base128 = s * (STAGES * K)
    my_rows = pl.ds(s * ROWS_PER_TILE, ROWS_PER_TILE)

    def issue_edges(st, bufs):
        row_r, col_r, val_r = bufs
        pltpu.async_copy(row_hbm.at[pl.ds(st, K)], row_r, esem)
        pltpu.async_copy(col_hbm.at[pl.ds(st, K)], col_r, esem)
        pltpu.async_copy(val_hbm.at[pl.ds(st, K)], val_r, esem)

    def drain_edges(st, bufs):
        row_r, col_r, val_r = bufs
        pltpu.make_async_copy(row_hbm.at[pl.ds(st, K)], row_r, esem).wait()
        pltpu.make_async_copy(col_hbm.at[pl.ds(st, K)], col_r, esem).wait()
        pltpu.make_async_copy(val_hbm.at[pl.ds(st, K)], val_r, esem).wait()

    def scale(j, val_r):
        mb = msg.at[j % 6]

        def scale_group(g, _):
            vv = val_r[j, pl.ds(g * 16, 16)]
            base = g * 16
            for e in range(16):
                mb[base + e, :] = mb[base + e, :] * vv[e]
            return 0

        lax.fori_loop(0, 8, scale_group, 0)

    def stage_block(src, bufs):
        row_r, col_r, val_r = bufs
        gsems = (gsem0, gsem1, gsem2)
        ssems = (ssem0, ssem1)
        g = [None] * K
        sc = [None] * K
        for p in range(3):
            g[p] = pltpu.async_copy(src.at[col_r.at[p]], msg.at[p], gsems[p])
        for j in range(K):
            g[j].wait()
            if j + 3 < K:
                g[j + 3] = pltpu.async_copy(
                    src.at[col_r.at[j + 3]], msg.at[(j + 3) % 6], gsems[j % 3])
            scale(j, val_r)
            if j >= 2:
                sc[j - 2].wait()
            sc[j] = pltpu.async_copy(
                msg.at[j % 6], acc_sh.at[row_r.at[j]], ssems[j % 2], add=True)
        sc[K - 2].wait()
        sc[K - 1].wait()

    bufs_a = (row_a, col_a, val_a)
    bufs_b = (row_b, col_b, val_b)

    pltpu.sync_copy(zeros_hbm, acc_sh.at[my_rows])
    issue_edges(base128, bufs_a)
    plsc.subcore_barrier()

    src = tab_hbm.at[c]

    def pair_body(t, _):
        st0 = base128 + (2 * t) * K
        st1 = st0 + K
        st2 = st1 + K
        drain_edges(st0, bufs_a)
        issue_edges(st1, bufs_b)
        stage_block(src, bufs_a)
        drain_edges(st1, bufs_b)

        @pl.when(t + 1 < STAGES // 2)
        def _():
            issue_edges(st2, bufs_a)

        stage_block(src, bufs_b)
        return 0

    lax.fori_loop(0, STAGES // 2, pair_body, 0)

    plsc.subcore_barrier()
    pltpu.sync_copy(acc_sh.at[my_rows], out_hbm.at[c, my_rows])


def kernel(user_embed, item_embed, edge_index, edge_vals):
    all_embed = jnp.concatenate([user_embed, item_embed], axis=0)
    all_embed = jnp.pad(all_embed, ((0, N_PAD - N_TOTAL), (0, 0)))
    tab = jnp.stack([all_embed[:, :HALF], all_embed[:, HALF:]])

    pad = E_PAD - N_EDGES
    row = jnp.concatenate([edge_index[0], jnp.zeros((pad,), edge_index.dtype)])
    col = jnp.concatenate([edge_index[1], jnp.zeros((pad,), edge_index.dtype)])
    val = jnp.concatenate([edge_vals, jnp.zeros((pad,), edge_vals.dtype)])
    row = row.reshape(-1, 128)
    col = col.reshape(-1, 128)
    val = val.reshape(-1, 128)
    zeros = jnp.zeros((ROWS_PER_TILE, HALF), jnp.float32)

    tabs = [tab]
    for _ in range(N_HOPS):
        tabs.append(_hop(tabs[-1], row, col, val, zeros))

    embs = jnp.stack(
        [jnp.concatenate([t[0, :N_TOTAL], t[1, :N_TOTAL]], axis=-1) for t in tabs],
        axis=1,
    )
    return embs[:N_USERS], embs[N_USERS:]


# speedup vs baseline: 1.3268x; 1.0741x over previous
import functools

import jax
import jax.numpy as jnp
from jax import lax
from jax.experimental import pallas as pl
from jax.experimental.pallas import tpu as pltpu
from jax.experimental.pallas import tpu_sc as plsc

N_USERS = 50000
N_ITEMS = 50000
N_TOTAL = N_USERS + N_ITEMS
EMB_DIM = 32
HALF = 16
N_EDGES = 1600000
N_HOPS = 3

NS = 16
K = 8
CHUNK = K * 128
STAGES = 98
EDGES_PER_TILE = STAGES * CHUNK
E_PAD = NS * EDGES_PER_TILE
N_PAD = 100096
ROWS_PER_TILE = N_PAD // NS

_mesh = plsc.VectorSubcoreMesh(core_axis_name="c", subcore_axis_name="s")


@functools.partial(
    pl.kernel,
    mesh=_mesh,
    out_type=jax.ShapeDtypeStruct((2, N_PAD, HALF), jnp.float32),
    compiler_params=pltpu.CompilerParams(use_tc_tiling_on_sc=False),
    scratch_types=[
        pltpu.VMEM((K, 128), jnp.int32),
        pltpu.VMEM((K, 128), jnp.int32),
        pltpu.VMEM((K, 128), jnp.float32),
        pltpu.VMEM((K, 128), jnp.int32),
        pltpu.VMEM((K, 128), jnp.int32),
        pltpu.VMEM((K, 128), jnp.float32),
        pltpu.VMEM((8, 128, HALF), jnp.float32),
        pltpu.SemaphoreType.DMA,
        pltpu.SemaphoreType.DMA,
        pltpu.SemaphoreType.DMA,
        pltpu.SemaphoreType.DMA,
        pltpu.SemaphoreType.DMA,
        pltpu.SemaphoreType.DMA,
        pltpu.SemaphoreType.DMA,
        pltpu.VMEM_SHARED((N_PAD, HALF), jnp.float32),
    ],
)
def _hop(tab_hbm, row_hbm, col_hbm, val_hbm, zeros_hbm, out_hbm,
         row_a, col_a, val_a, row_b, col_b, val_b, msg,
         esem, gsem0, gsem1, gsem2, gsem3, ssem0, ssem1, acc_sh):
    c = lax.axis_index("c")
    s = lax.axis_index("s")

    base128 = s * (STAGES * K)
    my_rows = pl.ds(s * ROWS_PER_TILE, ROWS_PER_TILE)

    def issue_edges(st, bufs):
        row_r, col_r, val_r = bufs
        pltpu.async_copy(row_hbm.at[pl.ds(st, K)], row_r, esem)
        pltpu.async_copy(col_hbm.at[pl.ds(st, K)], col_r, esem)
        pltpu.async_copy(val_hbm.at[pl.ds(st, K)], val_r, esem)

    def drain_edges(st, bufs):
        row_r, col_r, val_r = bufs
        pltpu.make_async_copy(row_hbm.at[pl.ds(st, K)], row_r, esem).wait()
        pltpu.make_async_copy(col_hbm.at[pl.ds(st, K)], col_r, esem).wait()
        pltpu.make_async_copy(val_hbm.at[pl.ds(st, K)], val_r, esem).wait()

    def scale(j, val_r):
        mb = msg.at[j % 8]

        def scale_group(g, _):
            vv = val_r[j, pl.ds(g * 16, 16)]
            base = g * 16
            for e in range(16):
                mb[base + e, :] = mb[base + e, :] * vv[e]
            return 0

        lax.fori_loop(0, 8, scale_group, 0)

    def stage_block(src, bufs):
        row_r, col_r, val_r = bufs
        gsems = (gsem0, gsem1, gsem2, gsem3)
        ssems = (ssem0, ssem1)
        g = [None] * K
        sc = [None] * K
        for p in range(4):
            g[p] = pltpu.async_copy(src.at[col_r.at[p]], msg.at[p], gsems[p])
        for j in range(K):
            g[j].wait()
            if j + 4 < K:
                g[j + 4] = pltpu.async_copy(
                    src.at[col_r.at[j + 4]], msg.at[(j + 4) % 8], gsems[j % 4])
            scale(j, val_r)
            if j >= 2:
                sc[j - 2].wait()
            sc[j] = pltpu.async_copy(
                msg.at[j % 8], acc_sh.at[row_r.at[j]], ssems[j % 2], add=True)
        sc[K - 2].wait()
        sc[K - 1].wait()

    bufs_a = (row_a, col_a, val_a)
    bufs_b = (row_b, col_b, val_b)

    pltpu.sync_copy(zeros_hbm, acc_sh.at[my_rows])
    issue_edges(base128, bufs_a)
    plsc.subcore_barrier()

    src = tab_hbm.at[c]

    def pair_body(t, _):
        st0 = base128 + (2 * t) * K
        st1 = st0 + K
        st2 = st1 + K
        drain_edges(st0, bufs_a)
        issue_edges(st1, bufs_b)
        stage_block(src, bufs_a)
        drain_edges(st1, bufs_b)

        @pl.when(t + 1 < STAGES // 2)
        def _():
            issue_edges(st2, bufs_a)

        stage_block(src, bufs_b)
        return 0

    lax.fori_loop(0, STAGES // 2, pair_body, 0)

    plsc.subcore_barrier()
    pltpu.sync_copy(acc_sh.at[my_rows], out_hbm.at[c, my_rows])


def kernel(user_embed, item_embed, edge_index, edge_vals):
    all_embed = jnp.concatenate([user_embed, item_embed], axis=0)
    all_embed = jnp.pad(all_embed, ((0, N_PAD - N_TOTAL), (0, 0)))
    tab = jnp.stack([all_embed[:, :HALF], all_embed[:, HALF:]])

    pad = E_PAD - N_EDGES
    row = jnp.concatenate([edge_index[0], jnp.zeros((pad,), edge_index.dtype)])
    col = jnp.concatenate([edge_index[1], jnp.zeros((pad,), edge_index.dtype)])
    val = jnp.concatenate([edge_vals, jnp.zeros((pad,), edge_vals.dtype)])
    row = row.reshape(-1, 128)
    col = col.reshape(-1, 128)
    val = val.reshape(-1, 128)
    zeros = jnp.zeros((ROWS_PER_TILE, HALF), jnp.float32)

    tabs = [tab]
    for _ in range(N_HOPS):
        tabs.append(_hop(tabs[-1], row, col, val, zeros))

    embs = jnp.stack(
        [jnp.concatenate([t[0, :N_TOTAL], t[1, :N_TOTAL]], axis=-1) for t in tabs],
        axis=1,
    )
    return embs[:N_USERS], embs[N_USERS:]


# speedup vs baseline: 1.4417x; 1.0866x over previous
import functools

import jax
import jax.numpy as jnp
from jax import lax
from jax.experimental import pallas as pl
from jax.experimental.pallas import tpu as pltpu
from jax.experimental.pallas import tpu_sc as plsc

N_USERS = 50000
N_ITEMS = 50000
N_TOTAL = N_USERS + N_ITEMS
EMB_DIM = 32
HALF = 16
N_EDGES = 1600000
N_HOPS = 3

NS = 16
K = 16
CHUNK = K * 128
STAGES = 50
EDGES_PER_TILE = STAGES * CHUNK
E_PAD = NS * EDGES_PER_TILE
N_PAD = 100096
ROWS_PER_TILE = N_PAD // NS

_mesh = plsc.VectorSubcoreMesh(core_axis_name="c", subcore_axis_name="s")


@functools.partial(
    pl.kernel,
    mesh=_mesh,
    out_type=jax.ShapeDtypeStruct((2, N_PAD, HALF), jnp.float32),
    compiler_params=pltpu.CompilerParams(use_tc_tiling_on_sc=False),
    scratch_types=[
        pltpu.VMEM((K, 128), jnp.int32),
        pltpu.VMEM((K, 128), jnp.int32),
        pltpu.VMEM((K, 128), jnp.float32),
        pltpu.VMEM((K, 128), jnp.int32),
        pltpu.VMEM((K, 128), jnp.int32),
        pltpu.VMEM((K, 128), jnp.float32),
        pltpu.VMEM((8, 128, HALF), jnp.float32),
        pltpu.SemaphoreType.DMA,
        pltpu.SemaphoreType.DMA,
        pltpu.SemaphoreType.DMA,
        pltpu.SemaphoreType.DMA,
        pltpu.SemaphoreType.DMA,
        pltpu.SemaphoreType.DMA,
        pltpu.SemaphoreType.DMA,
        pltpu.VMEM_SHARED((N_PAD, HALF), jnp.float32),
    ],
)
def _hop(tab_hbm, row_hbm, col_hbm, val_hbm, zeros_hbm, out_hbm,
         row_a, col_a, val_a, row_b, col_b, val_b, msg,
         esem, gsem0, gsem1, gsem2, gsem3, ssem0, ssem1, acc_sh):
    c = lax.axis_index("c")
    s = lax.axis_index("s")

    base128 = s * (STAGES * K)
    my_rows = pl.ds(s * ROWS_PER_TILE, ROWS_PER_TILE)

    def issue_edges(st, bufs):
        row_r, col_r, val_r = bufs
        pltpu.async_copy(row_hbm.at[pl.ds(st, K)], row_r, esem)
        pltpu.async_copy(col_hbm.at[pl.ds(st, K)], col_r, esem)
        pltpu.async_copy(val_hbm.at[pl.ds(st, K)], val_r, esem)

    def drain_edges(st, bufs):
        row_r, col_r, val_r = bufs
        pltpu.make_async_copy(row_hbm.at[pl.ds(st, K)], row_r, esem).wait()
        pltpu.make_async_copy(col_hbm.at[pl.ds(st, K)], col_r, esem).wait()
        pltpu.make_async_copy(val_hbm.at[pl.ds(st, K)], val_r, esem).wait()

    def scale(j, val_r):
        mb = msg.at[j % 8]

        def scale_group(g, _):
            vv = val_r[j, pl.ds(g * 16, 16)]
            base = g * 16
            for e in range(16):
                mb[base + e, :] = mb[base + e, :] * vv[e]
            return 0

        lax.fori_loop(0, 8, scale_group, 0)

    def stage_block(src, bufs):
        row_r, col_r, val_r = bufs
        gsems = (gsem0, gsem1, gsem2, gsem3)
        ssems = (ssem0, ssem1)
        g = [None] * K
        sc = [None] * K
        for p in range(4):
            g[p] = pltpu.async_copy(src.at[col_r.at[p]], msg.at[p], gsems[p])
        for j in range(K):
            g[j].wait()
            if j + 4 < K:
                g[j + 4] = pltpu.async_copy(
                    src.at[col_r.at[j + 4]], msg.at[(j + 4) % 8], gsems[j % 4])
            scale(j, val_r)
            if j >= 2:
                sc[j - 2].wait()
            sc[j] = pltpu.async_copy(
                msg.at[j % 8], acc_sh.at[row_r.at[j]], ssems[j % 2], add=True)
        sc[K - 2].wait()
        sc[K - 1].wait()

    bufs_a = (row_a, col_a, val_a)
    bufs_b = (row_b, col_b, val_b)

    pltpu.sync_copy(zeros_hbm, acc_sh.at[my_rows])
    issue_edges(base128, bufs_a)
    plsc.subcore_barrier()

    src = tab_hbm.at[c]

    def pair_body(t, _):
        st0 = base128 + (2 * t) * K
        st1 = st0 + K
        st2 = st1 + K
        drain_edges(st0, bufs_a)
        issue_edges(st1, bufs_b)
        stage_block(src, bufs_a)
        drain_edges(st1, bufs_b)

        @pl.when(t + 1 < STAGES // 2)
        def _():
            issue_edges(st2, bufs_a)

        stage_block(src, bufs_b)
        return 0

    lax.fori_loop(0, STAGES // 2, pair_body, 0)

    plsc.subcore_barrier()
    pltpu.sync_copy(acc_sh.at[my_rows], out_hbm.at[c, my_rows])


def kernel(user_embed, item_embed, edge_index, edge_vals):
    all_embed = jnp.concatenate([user_embed, item_embed], axis=0)
    all_embed = jnp.pad(all_embed, ((0, N_PAD - N_TOTAL), (0, 0)))
    tab = jnp.stack([all_embed[:, :HALF], all_embed[:, HALF:]])

    pad = E_PAD - N_EDGES
    pad_ids = (jnp.arange(pad, dtype=edge_index.dtype) * 97) % N_TOTAL
    row = jnp.concatenate([edge_index[0], pad_ids])
    col = jnp.concatenate([edge_index[1], pad_ids])
    val = jnp.concatenate([edge_vals, jnp.zeros((pad,), edge_vals.dtype)])
    row = row.reshape(-1, 128)
    col = col.reshape(-1, 128)
    val = val.reshape(-1, 128)
    zeros = jnp.zeros((ROWS_PER_TILE, HALF), jnp.float32)

    tabs = [tab]
    for _ in range(N_HOPS):
        tabs.append(_hop(tabs[-1], row, col, val, zeros))

    embs = jnp.stack(
        [jnp.concatenate([t[0, :N_TOTAL], t[1, :N_TOTAL]], axis=-1) for t in tabs],
        axis=1,
    )
    return embs[:N_USERS], embs[N_USERS:]


# speedup vs baseline: 1.5203x; 1.0545x over previous
import functools

import jax
import jax.numpy as jnp
from jax import lax
from jax.experimental import pallas as pl
from jax.experimental.pallas import tpu as pltpu
from jax.experimental.pallas import tpu_sc as plsc

N_USERS = 50000
N_ITEMS = 50000
N_TOTAL = N_USERS + N_ITEMS
EMB_DIM = 32
HALF = 16
N_EDGES = 1600000
N_HOPS = 3

NS = 16
K = 16
CHUNK = K * 128
STAGES = 50
EDGES_PER_TILE = STAGES * CHUNK
E_PAD = NS * EDGES_PER_TILE
N_PAD = 100096
ROWS_PER_TILE = N_PAD // NS

_mesh = plsc.VectorSubcoreMesh(core_axis_name="c", subcore_axis_name="s")


@functools.partial(
    pl.kernel,
    mesh=_mesh,
    out_type=jax.ShapeDtypeStruct((2, N_PAD, HALF), jnp.float32),
    compiler_params=pltpu.CompilerParams(use_tc_tiling_on_sc=False),
    scratch_types=[
        pltpu.VMEM((K, 128), jnp.int32),
        pltpu.VMEM((K, 128), jnp.int32),
        pltpu.VMEM((K, 128), jnp.float32),
        pltpu.VMEM((K, 128), jnp.int32),
        pltpu.VMEM((K, 128), jnp.int32),
        pltpu.VMEM((K, 128), jnp.float32),
        pltpu.VMEM((9, 128, HALF), jnp.float32),
        pltpu.SemaphoreType.DMA,
        pltpu.SemaphoreType.DMA,
        pltpu.SemaphoreType.DMA,
        pltpu.SemaphoreType.DMA,
        pltpu.SemaphoreType.DMA,
        pltpu.SemaphoreType.DMA,
        pltpu.SemaphoreType.DMA,
        pltpu.SemaphoreType.DMA,
        pltpu.SemaphoreType.DMA,
        pltpu.VMEM_SHARED((N_PAD, HALF), jnp.float32),
    ],
)
def _hop(tab_hbm, row_hbm, col_hbm, val_hbm, zeros_hbm, out_hbm,
         row_a, col_a, val_a, row_b, col_b, val_b, msg,
         esem, gsem0, gsem1, gsem2, gsem3, gsem4, gsem5, ssem0, ssem1, acc_sh):
    c = lax.axis_index("c")
    s = lax.axis_index("s")

    base128 = s * (STAGES * K)
    my_rows = pl.ds(s * ROWS_PER_TILE, ROWS_PER_TILE)

    def issue_edges(st, bufs):
        row_r, col_r, val_r = bufs
        pltpu.async_copy(row_hbm.at[pl.ds(st, K)], row_r, esem)
        pltpu.async_copy(col_hbm.at[pl.ds(st, K)], col_r, esem)
        pltpu.async_copy(val_hbm.at[pl.ds(st, K)], val_r, esem)

    def drain_edges(st, bufs):
        row_r, col_r, val_r = bufs
        pltpu.make_async_copy(row_hbm.at[pl.ds(st, K)], row_r, esem).wait()
        pltpu.make_async_copy(col_hbm.at[pl.ds(st, K)], col_r, esem).wait()
        pltpu.make_async_copy(val_hbm.at[pl.ds(st, K)], val_r, esem).wait()

    def scale(j, val_r):
        mb = msg.at[j % 9]

        def scale_group(g, _):
            vv = val_r[j, pl.ds(g * 16, 16)]
            base = g * 16
            for e in range(16):
                mb[base + e, :] = mb[base + e, :] * vv[e]
            return 0

        lax.fori_loop(0, 8, scale_group, 0)

    def stage_block(src, bufs):
        row_r, col_r, val_r = bufs
        gsems = (gsem0, gsem1, gsem2, gsem3, gsem4, gsem5)
        ssems = (ssem0, ssem1)
        g = [None] * K
        sc = [None] * K
        for p in range(6):
            g[p] = pltpu.async_copy(src.at[col_r.at[p]], msg.at[p], gsems[p])
        for j in range(K):
            g[j].wait()
            if j + 6 < K:
                g[j + 6] = pltpu.async_copy(
                    src.at[col_r.at[j + 6]], msg.at[(j + 6) % 9], gsems[j % 6])
            scale(j, val_r)
            if j >= 2:
                sc[j - 2].wait()
            sc[j] = pltpu.async_copy(
                msg.at[j % 9], acc_sh.at[row_r.at[j]], ssems[j % 2], add=True)
        sc[K - 2].wait()
        sc[K - 1].wait()

    bufs_a = (row_a, col_a, val_a)
    bufs_b = (row_b, col_b, val_b)

    pltpu.sync_copy(zeros_hbm, acc_sh.at[my_rows])
    issue_edges(base128, bufs_a)
    plsc.subcore_barrier()

    src = tab_hbm.at[c]

    def pair_body(t, _):
        st0 = base128 + (2 * t) * K
        st1 = st0 + K
        st2 = st1 + K
        drain_edges(st0, bufs_a)
        issue_edges(st1, bufs_b)
        stage_block(src, bufs_a)
        drain_edges(st1, bufs_b)

        @pl.when(t + 1 < STAGES // 2)
        def _():
            issue_edges(st2, bufs_a)

        stage_block(src, bufs_b)
        return 0

    lax.fori_loop(0, STAGES // 2, pair_body, 0)

    plsc.subcore_barrier()
    pltpu.sync_copy(acc_sh.at[my_rows], out_hbm.at[c, my_rows])


def kernel(user_embed, item_embed, edge_index, edge_vals):
    all_embed = jnp.concatenate([user_embed, item_embed], axis=0)
    all_embed = jnp.pad(all_embed, ((0, N_PAD - N_TOTAL), (0, 0)))
    tab = jnp.stack([all_embed[:, :HALF], all_embed[:, HALF:]])

    pad = E_PAD - N_EDGES
    pad_ids = (jnp.arange(pad, dtype=edge_index.dtype) * 97) % N_TOTAL
    row = jnp.concatenate([edge_index[0], pad_ids])
    col = jnp.concatenate([edge_index[1], pad_ids])
    val = jnp.concatenate([edge_vals, jnp.zeros((pad,), edge_vals.dtype)])
    row = row.reshape(-1, 128)
    col = col.reshape(-1, 128)
    val = val.reshape(-1, 128)
    zeros = jnp.zeros((ROWS_PER_TILE, HALF), jnp.float32)

    tabs = [tab]
    for _ in range(N_HOPS):
        tabs.append(_hop(tabs[-1], row, col, val, zeros))

    embs = jnp.stack(
        [jnp.concatenate([t[0, :N_TOTAL], t[1, :N_TOTAL]], axis=-1) for t in tabs],
        axis=1,
    )
    return embs[:N_USERS], embs[N_USERS:]
